# baseline (speedup 1.0000x reference)
import functools

import jax
import jax.numpy as jnp
from jax import lax
from jax.experimental import pallas as pl
from jax.experimental.pallas import tpu as pltpu, tpu_sc as plsc


def _make_gather(n_rows: int, n_idx: int, d: int):
    info = plsc.get_sparse_core_info()
    nc, ns = info.num_cores, info.num_subcores
    nw = nc * ns
    assert n_idx % nw == 0
    per_w = n_idx // nw
    chunk = 32
    nbuf = 3
    n_chunks = per_w // chunk
    mesh = plsc.VectorSubcoreMesh(core_axis_name="c", subcore_axis_name="s")

    @functools.partial(
        pl.kernel,
        mesh=mesh,
        out_type=jax.ShapeDtypeStruct((n_idx, d), jnp.float32),
        scratch_types=[
            pltpu.VMEM((per_w,), jnp.int32),
            pltpu.VMEM((nbuf, chunk, d), jnp.float32),
            pltpu.SemaphoreType.DMA,
            pltpu.SemaphoreType.DMA,
        ],
    )
    def gather_k(table_hbm, idx_hbm, out_hbm, idx_v, rows_v, gsem, wsem):
        wid = lax.axis_index("s") * nc + lax.axis_index("c")
        base = wid * per_w
        pltpu.sync_copy(idx_hbm.at[pl.ds(base, per_w)], idx_v)

        def start_gather(j):
            return pltpu.async_copy(
                table_hbm.at[idx_v.at[pl.ds(j * chunk, chunk)]],
                rows_v.at[j % nbuf], gsem)

        gathers = [None] * n_chunks
        writes = [None] * n_chunks
        for j in range(min(nbuf, n_chunks)):
            gathers[j] = start_gather(j)
        for i in range(n_chunks):
            gathers[i].wait()
            writes[i] = pltpu.async_copy(
                rows_v.at[i % nbuf],
                out_hbm.at[pl.ds(base + i * chunk, chunk)], wsem)
            nxt = i + nbuf
            if nxt < n_chunks:
                writes[i].wait()
                gathers[nxt] = start_gather(nxt)
        for i in range(max(0, n_chunks - nbuf), n_chunks):
            writes[i].wait()

    return gather_k


def kernel(hidden, word_absolute_position):
    B, S, D = hidden.shape
    table = hidden.reshape(B * S, D)
    idx = word_absolute_position.astype(jnp.int32)
    n_idx = idx.shape[0]
    out = _make_gather(B * S, n_idx, D)(table, idx)
    return out.reshape(n_idx // 2, 2 * D)


# speedup vs baseline: 1.7496x; 1.7496x over previous
import functools

import jax
import jax.numpy as jnp
from jax import lax
from jax.experimental import pallas as pl
from jax.experimental.pallas import tpu as pltpu, tpu_sc as plsc


def _make_gather(n_out: int, d: int):
    info = plsc.get_sparse_core_info()
    nc, ns, nl = info.num_cores, info.num_subcores, info.num_lanes
    nw = nc * ns
    assert n_out % nw == 0
    per_w = n_out // nw
    chunk = 16
    nbuf = 3
    n_chunks = per_w // chunk
    mesh = plsc.VectorSubcoreMesh(core_axis_name="c", subcore_axis_name="s")

    @functools.partial(
        pl.kernel,
        mesh=mesh,
        out_type=jax.ShapeDtypeStruct((n_out, 2 * d), jnp.float32),
        scratch_types=[
            pltpu.VMEM((per_w,), jnp.int32),
            pltpu.VMEM((per_w,), jnp.int32),
            pltpu.VMEM((nbuf, chunk, d), jnp.float32),
            pltpu.VMEM((nbuf, chunk, d), jnp.float32),
            pltpu.SemaphoreType.DMA,
            pltpu.SemaphoreType.DMA,
        ],
    )
    def gather_k(table_hbm, idx_e_hbm, idx_o_hbm, out_hbm,
                 idx_ev, idx_ov, buf_e, buf_o, gsem, wsem):
        wid = lax.axis_index("s") * nc + lax.axis_index("c")
        base = wid * per_w
        pltpu.sync_copy(idx_e_hbm.at[pl.ds(base, per_w)], idx_ev)
        pltpu.sync_copy(idx_o_hbm.at[pl.ds(base, per_w)], idx_ov)

        def start_gathers(j):
            b = j % nbuf
            ge = pltpu.async_copy(
                table_hbm.at[idx_ev.at[pl.ds(j * chunk, chunk)]],
                buf_e.at[b], gsem)
            go = pltpu.async_copy(
                table_hbm.at[idx_ov.at[pl.ds(j * chunk, chunk)]],
                buf_o.at[b], gsem)
            return ge, go

        gathers = [None] * n_chunks
        writes = [None] * n_chunks
        for j in range(min(nbuf, n_chunks)):
            gathers[j] = start_gathers(j)
        for i in range(n_chunks):
            b = i % nbuf
            ge, go = gathers[i]
            ge.wait()
            go.wait()
            row = base + i * chunk
            we = pltpu.async_copy(
                buf_e.at[b], out_hbm.at[pl.ds(row, chunk), pl.ds(0, d)], wsem)
            wo = pltpu.async_copy(
                buf_o.at[b], out_hbm.at[pl.ds(row, chunk), pl.ds(d, d)], wsem)
            writes[i] = (we, wo)
            nxt = i + nbuf
            if nxt < n_chunks:
                we.wait()
                wo.wait()
                gathers[nxt] = start_gathers(nxt)
        for i in range(max(0, n_chunks - nbuf), n_chunks):
            we, wo = writes[i]
            we.wait()
            wo.wait()

    return gather_k


def kernel(hidden, word_absolute_position):
    B, S, D = hidden.shape
    table = hidden.reshape(B * S, D)
    idx = word_absolute_position.astype(jnp.int32)
    n_idx = idx.shape[0]
    idx2 = idx.reshape(n_idx // 2, 2)
    return _make_gather(n_idx // 2, D)(table, idx2[:, 0], idx2[:, 1])


# speedup vs baseline: 1.8498x; 1.0573x over previous
import functools

import jax
import jax.numpy as jnp
from jax import lax
from jax.experimental import pallas as pl
from jax.experimental.pallas import tpu as pltpu, tpu_sc as plsc


def _make_gather(n_out: int, d: int):
    info = plsc.get_sparse_core_info()
    nc, ns, nl = info.num_cores, info.num_subcores, info.num_lanes
    nw = nc * ns
    assert n_out % nw == 0
    per_w = n_out // nw
    chunk = 16
    nbuf = 3
    n_chunks = per_w // chunk
    mesh = plsc.VectorSubcoreMesh(core_axis_name="c", subcore_axis_name="s")

    @functools.partial(
        pl.kernel,
        mesh=mesh,
        out_type=jax.ShapeDtypeStruct((n_out, 2 * d), jnp.float32),
        scratch_types=[
            pltpu.VMEM((2 * per_w,), jnp.int32),
            pltpu.VMEM((per_w,), jnp.int32),
            pltpu.VMEM((per_w,), jnp.int32),
            pltpu.VMEM((nbuf, chunk, d), jnp.float32),
            pltpu.VMEM((nbuf, chunk, d), jnp.float32),
            pltpu.SemaphoreType.DMA,
            pltpu.SemaphoreType.DMA,
        ],
    )
    def gather_k(table_hbm, idx_hbm, out_hbm,
                 idx_v, idx_ev, idx_ov, buf_e, buf_o, gsem, wsem):
        wid = lax.axis_index("s") * nc + lax.axis_index("c")
        base = wid * per_w
        pltpu.sync_copy(idx_hbm.at[pl.ds(2 * base, 2 * per_w)], idx_v)

        lanes = lax.iota(jnp.int32, nl)
        g_e = (2 * lanes) % nl
        g_o = (2 * lanes + 1) % nl
        lo = lanes < (nl // 2)
        for j in range(per_w // nl):
            a = idx_v[pl.ds(2 * nl * j, nl)]
            b = idx_v[pl.ds(2 * nl * j + nl, nl)]
            idx_ev[pl.ds(nl * j, nl)] = jnp.where(
                lo,
                a.at[g_e].get(mode="promise_in_bounds"),
                b.at[g_e].get(mode="promise_in_bounds"))
            idx_ov[pl.ds(nl * j, nl)] = jnp.where(
                lo,
                a.at[g_o].get(mode="promise_in_bounds"),
                b.at[g_o].get(mode="promise_in_bounds"))

        def start_gathers(j):
            b = j % nbuf
            ge = pltpu.async_copy(
                table_hbm.at[idx_ev.at[pl.ds(j * chunk, chunk)]],
                buf_e.at[b], gsem)
            go = pltpu.async_copy(
                table_hbm.at[idx_ov.at[pl.ds(j * chunk, chunk)]],
                buf_o.at[b], gsem)
            return ge, go

        gathers = [None] * n_chunks
        writes = [None] * n_chunks
        for j in range(min(nbuf, n_chunks)):
            gathers[j] = start_gathers(j)
        for i in range(n_chunks):
            b = i % nbuf
            for g in gathers[i]:
                g.wait()
            row = base + i * chunk
            we = pltpu.async_copy(
                buf_e.at[b], out_hbm.at[pl.ds(row, chunk), pl.ds(0, d)], wsem)
            wo = pltpu.async_copy(
                buf_o.at[b], out_hbm.at[pl.ds(row, chunk), pl.ds(d, d)], wsem)
            writes[i] = (we, wo)
            nxt = i + nbuf
            if nxt < n_chunks:
                we.wait()
                wo.wait()
                gathers[nxt] = start_gathers(nxt)
        for i in range(max(0, n_chunks - nbuf), n_chunks):
            for w in writes[i]:
                w.wait()

    return gather_k


def kernel(hidden, word_absolute_position):
    B, S, D = hidden.shape
    table = hidden.reshape(B * S, D)
    idx = word_absolute_position.astype(jnp.int32)
    n_idx = idx.shape[0]
    return _make_gather(n_idx // 2, D)(table, idx)
